# baseline (speedup 1.0000x reference)
import functools

import jax
import jax.numpy as jnp
from jax import lax
from jax.experimental import pallas as pl
from jax.experimental.pallas import tpu as pltpu
from jax.experimental.pallas import tpu_sc as plsc

D = 64
CHUNK = 800
NBUF = 2
NC = 2
NS = 16
NW = NC * NS


def kernel(x, pe):
    B = x.size
    assert B % (NW * NBUF * CHUNK) == 0
    n_bodies = B // (NW * NBUF * CHUNK)
    xf = x.reshape(B)
    per_w = B // NW

    mesh = plsc.VectorSubcoreMesh(
        core_axis_name="c", subcore_axis_name="s", num_cores=NC, num_subcores=NS
    )

    @functools.partial(
        pl.kernel,
        mesh=mesh,
        compiler_params=pltpu.CompilerParams(use_tc_tiling_on_sc=False),
        out_type=jax.ShapeDtypeStruct((B, D), jnp.float32),
        scratch_types=[
            pltpu.VMEM((NBUF, CHUNK), jnp.int32),
            pltpu.VMEM((NBUF, CHUNK, D), jnp.float32),
            pltpu.SemaphoreType.DMA,
            pltpu.SemaphoreType.DMA,
            pltpu.SemaphoreType.DMA,
            pltpu.SemaphoreType.DMA,
            pltpu.SemaphoreType.DMA,
            pltpu.SemaphoreType.DMA,
        ],
    )
    def gather_kernel(idx_hbm, table_hbm, out_hbm, idx_v, rows_v,
                      gsem0, gsem1, osem0, osem1, isem0, isem1):
        wid = lax.axis_index("s") * NC + lax.axis_index("c")
        base = wid * per_w
        gsems = (gsem0, gsem1)
        osems = (osem0, osem1)
        isems = (isem0, isem1)

        def fire_idx(r0, b):
            pltpu.async_copy(idx_hbm.at[pl.ds(r0, CHUNK)], idx_v.at[b], isems[b])

        def drain_idx(b):
            pltpu.make_async_copy(
                idx_hbm.at[pl.ds(0, CHUNK)], idx_v.at[b], isems[b]
            ).wait()

        def fire_gather(b):
            return pltpu.async_copy(
                table_hbm.at[idx_v.at[b]], rows_v.at[b], gsems[b]
            )

        def drain_out(b):
            pltpu.make_async_copy(
                rows_v.at[b], out_hbm.at[pl.ds(0, CHUNK)], osems[b]
            ).wait()

        fire_idx(base, 0)
        fire_idx(base + CHUNK, 1)

        def body(g, carry):
            r0 = base + g * (NBUF * CHUNK)
            r1 = r0 + CHUNK
            nxt = lax.select(g + 1 < n_bodies, r0 + NBUF * CHUNK, base)

            @pl.when(g > 0)
            def _():
                drain_out(0)

            drain_idx(0)
            d0 = fire_gather(0)

            @pl.when(g > 0)
            def _():
                drain_out(1)

            drain_idx(1)
            d1 = fire_gather(1)
            d0.wait()
            fire_idx(nxt, 0)
            pltpu.async_copy(rows_v.at[0], out_hbm.at[pl.ds(r0, CHUNK)], osems[0])
            d1.wait()
            fire_idx(nxt + CHUNK, 1)
            pltpu.async_copy(rows_v.at[1], out_hbm.at[pl.ds(r1, CHUNK)], osems[1])
            return carry

        lax.fori_loop(0, n_bodies, body, 0)
        drain_idx(0)
        drain_idx(1)
        drain_out(0)
        drain_out(1)

    out = gather_kernel(xf, pe)
    return out.reshape(x.shape + (D,))


# speedup vs baseline: 1.0028x; 1.0028x over previous
import functools

import jax
import jax.numpy as jnp
from jax import lax
from jax.experimental import pallas as pl
from jax.experimental.pallas import tpu as pltpu
from jax.experimental.pallas import tpu_sc as plsc

D = 64
CHUNK = 800
NBUF = 2
NC = 2
NS = 16
NW = NC * NS


def kernel(x, pe):
    B = x.size
    assert B % (NW * NBUF * CHUNK) == 0
    n_bodies = B // (NW * NBUF * CHUNK)
    xf = x.reshape(B)
    per_w = B // NW

    mesh = plsc.VectorSubcoreMesh(
        core_axis_name="c", subcore_axis_name="s", num_cores=NC, num_subcores=NS
    )

    @functools.partial(
        pl.kernel,
        mesh=mesh,
        compiler_params=pltpu.CompilerParams(use_tc_tiling_on_sc=False),
        out_type=jax.ShapeDtypeStruct((B, D), jnp.float32),
        scratch_types=[
            pltpu.VMEM((NBUF, CHUNK), jnp.int32),
            pltpu.VMEM((NBUF, CHUNK, D), jnp.float32),
            pltpu.SemaphoreType.DMA,
            pltpu.SemaphoreType.DMA,
            pltpu.SemaphoreType.DMA,
            pltpu.SemaphoreType.DMA,
        ],
    )
    def gather_kernel(idx_hbm, table_hbm, out_hbm, idx_v, rows_v,
                      gsem0, gsem1, osem0, osem1):
        wid = lax.axis_index("s") * NC + lax.axis_index("c")
        base = wid * per_w
        gsems = (gsem0, gsem1)
        osems = (osem0, osem1)

        def fire_gather(r0, b):
            pltpu.sync_copy(idx_hbm.at[pl.ds(r0, CHUNK)], idx_v.at[b])
            return pltpu.async_copy(
                table_hbm.at[idx_v.at[b]], rows_v.at[b], gsems[b]
            )

        def drain_out(b):
            pltpu.make_async_copy(
                rows_v.at[b], out_hbm.at[pl.ds(0, CHUNK)], osems[b]
            ).wait()

        def body(g, carry):
            r0 = base + g * (NBUF * CHUNK)
            r1 = r0 + CHUNK

            @pl.when(g > 0)
            def _():
                drain_out(0)

            d0 = fire_gather(r0, 0)

            @pl.when(g > 0)
            def _():
                drain_out(1)

            d1 = fire_gather(r1, 1)
            d0.wait()
            pltpu.async_copy(rows_v.at[0], out_hbm.at[pl.ds(r0, CHUNK)], osems[0])
            d1.wait()
            pltpu.async_copy(rows_v.at[1], out_hbm.at[pl.ds(r1, CHUNK)], osems[1])
            return carry

        lax.fori_loop(0, n_bodies, body, 0)
        drain_out(0)
        drain_out(1)

    out = gather_kernel(xf, pe)
    return out.reshape(x.shape + (D,))
